# Initial kernel scaffold; baseline (speedup 1.0000x reference)
import jax
import jax.numpy as jnp
from jax.experimental import pallas as pl


def kernel(x, edge_index, batch, W1, b1, g1, be1, W2, b2, g2, be2, fcW, fcb):
    raise NotImplementedError("write your pallas kernel here")


# speedup vs baseline: 15.2263x; 15.2263x over previous
import functools

_IT = False

import jax
import jax.numpy as jnp
from jax import lax
from jax.experimental import pallas as pl
from jax.experimental.pallas import tpu as pltpu
from jax.experimental.pallas import tpu_sc as plsc

_N = 10000
_E = 320000
_DIN = 128
_DH = 256
_DHF = 128
_NB = 64
_NC = 2
_NS = 16
_K = 125
_AGG_CH = _E // (_NS * _K)
_DEG_CH = _E // (_NC * _NS * _K)
_HCH = _AGG_CH // 2
_RPT = 624
_TAIL0 = _RPT * _NS
_TAILN = _N - _TAIL0

_R = 1000
_G = _N // _R

_mesh = plsc.VectorSubcoreMesh(core_axis_name="c", subcore_axis_name="s",
                               num_cores=_NC, num_subcores=_NS)


def _deg_body(e_dst, zvec, out, idx_v, ones_v, tmp_v, acc):
    c = lax.axis_index("c")
    s = lax.axis_index("s")
    wid = s * _NC + c
    pltpu.sync_copy(e_dst.at[wid], idx_v)
    for i in range(8):
        ones_v[pl.ds(i * 16, 16)] = jnp.ones((16,), jnp.float32)

    @pl.when(s == 0)
    def _():
        pltpu.sync_copy(zvec, tmp_v)
        pltpu.sync_copy(tmp_v, acc)

    plsc.subcore_barrier()

    def step(j, carry):
        pltpu.sync_copy(ones_v.at[pl.ds(0, _K)], acc.at[idx_v.at[j]], add=True)
        return carry

    lax.fori_loop(0, _DEG_CH, step, 0)
    plsc.subcore_barrier()

    @pl.when(s == 0)
    def _():
        pltpu.sync_copy(acc, tmp_v)
        pltpu.sync_copy(tmp_v, out.at[pl.ds(c * _N, _N)])


_deg_call = pl.kernel(
    _deg_body,
    out_type=jax.ShapeDtypeStruct((_NC * _N,), jnp.float32),
    mesh=_mesh,
    interpret=_IT,
    scratch_types=[
        pltpu.VMEM((_DEG_CH, _K), jnp.int32),
        pltpu.VMEM((128,), jnp.float32),
        pltpu.VMEM((_N,), jnp.float32),
        pltpu.VMEM_SHARED((_N,), jnp.float32),
    ],
)


def _agg_body(hpA, hpB, e_src, e_dst, zrows, outA, outB,
              sidx, didx, rbuf, acc, gsem):
    c = lax.axis_index("c")
    s = lax.axis_index("s")
    row0 = s * _RPT
    pltpu.sync_copy(zrows.at[pl.ds(row0, _RPT)], acc.at[pl.ds(row0, _RPT)])

    @pl.when(s == _NS - 1)
    def _():
        pltpu.sync_copy(zrows.at[pl.ds(_TAIL0, _TAILN)],
                        acc.at[pl.ds(_TAIL0, _TAILN)])

    plsc.subcore_barrier()

    def run(table):
        def step(j, carry):
            pltpu.async_copy(table.at[sidx.at[j]], rbuf, gsem).wait()
            pltpu.sync_copy(rbuf, acc.at[didx.at[j]], add=True)
            return carry
        for p in range(2):
            pltpu.sync_copy(e_src.at[s, pl.ds(p * _HCH, _HCH)], sidx)
            pltpu.sync_copy(e_dst.at[s, pl.ds(p * _HCH, _HCH)], didx)
            lax.fori_loop(0, _HCH, step, 0)

    @pl.when(c == 0)
    def _():
        run(hpA)

    @pl.when(c == 1)
    def _():
        run(hpB)

    plsc.subcore_barrier()

    def copy_out(outref):
        pltpu.sync_copy(acc.at[pl.ds(row0, _RPT)], outref.at[pl.ds(row0, _RPT)])

        @pl.when(s == _NS - 1)
        def _():
            pltpu.sync_copy(acc.at[pl.ds(_TAIL0, _TAILN)],
                            outref.at[pl.ds(_TAIL0, _TAILN)])

    @pl.when(c == 0)
    def _():
        copy_out(outA)

    @pl.when(c == 1)
    def _():
        copy_out(outB)


_agg_call = pl.kernel(
    _agg_body,
    out_type=(jax.ShapeDtypeStruct((_N, _DHF), jnp.float32),
              jax.ShapeDtypeStruct((_N, _DHF), jnp.float32)),
    mesh=_mesh,
    interpret=_IT,
    scratch_types=[
        pltpu.VMEM((_HCH, _K), jnp.int32),
        pltpu.VMEM((_HCH, _K), jnp.int32),
        pltpu.VMEM((_K, _DHF), jnp.float32),
        pltpu.VMEM_SHARED((_N, _DHF), jnp.float32),
        pltpu.SemaphoreType.DMA,
    ],
)


def _lin1_body(x_ref, w1_ref, degT_ref, h1_ref, hpA_ref, hpB_ref, dis_ref):
    deg = degT_ref[:, 0:1] + degT_ref[:, 1:2] + 1.0
    dis = lax.rsqrt(deg)
    h1 = jnp.dot(x_ref[...], w1_ref[...], preferred_element_type=jnp.float32)
    hp = h1 * dis
    h1_ref[...] = h1
    hpA_ref[...] = hp[:, :_DHF]
    hpB_ref[...] = hp[:, _DHF:]
    dis_ref[...] = dis


_lin1_call = pl.pallas_call(
    _lin1_body,
    interpret=_IT,
    grid=(_G,),
    in_specs=[
        pl.BlockSpec((_R, _DIN), lambda i: (i, 0)),
        pl.BlockSpec((_DIN, _DH), lambda i: (0, 0)),
        pl.BlockSpec((_R, 2), lambda i: (i, 0)),
    ],
    out_specs=[
        pl.BlockSpec((_R, _DH), lambda i: (i, 0)),
        pl.BlockSpec((_R, _DHF), lambda i: (i, 0)),
        pl.BlockSpec((_R, _DHF), lambda i: (i, 0)),
        pl.BlockSpec((_R, 1), lambda i: (i, 0)),
    ],
    out_shape=(
        jax.ShapeDtypeStruct((_N, _DH), jnp.float32),
        jax.ShapeDtypeStruct((_N, _DHF), jnp.float32),
        jax.ShapeDtypeStruct((_N, _DHF), jnp.float32),
        jax.ShapeDtypeStruct((_N, 1), jnp.float32),
    ),
)


def _stats_body(aggA_ref, aggB_ref, hself_ref, dis_ref, b_ref,
                o_ref, psum_ref, psq_ref):
    i = pl.program_id(0)
    dis = dis_ref[...]
    agg = jnp.concatenate([aggA_ref[...], aggB_ref[...]], axis=1)
    o = dis * agg + (dis * dis) * hself_ref[...] + b_ref[...]
    o_ref[...] = o

    @pl.when(i == 0)
    def _():
        psum_ref[...] = jnp.zeros_like(psum_ref)
        psq_ref[...] = jnp.zeros_like(psq_ref)

    psum_ref[...] += jnp.sum(o, axis=0, keepdims=True)
    psq_ref[...] += jnp.sum(o * o, axis=0, keepdims=True)


_stats_call = pl.pallas_call(
    _stats_body,
    interpret=_IT,
    grid=(_G,),
    in_specs=[
        pl.BlockSpec((_R, _DHF), lambda i: (i, 0)),
        pl.BlockSpec((_R, _DHF), lambda i: (i, 0)),
        pl.BlockSpec((_R, _DH), lambda i: (i, 0)),
        pl.BlockSpec((_R, 1), lambda i: (i, 0)),
        pl.BlockSpec((1, _DH), lambda i: (0, 0)),
    ],
    out_specs=[
        pl.BlockSpec((_R, _DH), lambda i: (i, 0)),
        pl.BlockSpec((1, _DH), lambda i: (0, 0)),
        pl.BlockSpec((1, _DH), lambda i: (0, 0)),
    ],
    out_shape=(
        jax.ShapeDtypeStruct((_N, _DH), jnp.float32),
        jax.ShapeDtypeStruct((1, _DH), jnp.float32),
        jax.ShapeDtypeStruct((1, _DH), jnp.float32),
    ),
)


def _bn_relu(o, psum, psq, g, be):
    mu = psum * (1.0 / _N)
    var = psq * (1.0 / _N) - mu * mu
    return jnp.maximum(g * (o - mu) * lax.rsqrt(var + 1e-5) + be, 0.0)


def _mid_body(o_ref, psum_ref, psq_ref, g_ref, be_ref, w2_ref, dis_ref,
              h2_ref, h2pA_ref, h2pB_ref):
    h = _bn_relu(o_ref[...], psum_ref[...], psq_ref[...], g_ref[...], be_ref[...])
    h2 = jnp.dot(h, w2_ref[...], preferred_element_type=jnp.float32)
    h2_ref[...] = h2
    hp = h2 * dis_ref[...]
    h2pA_ref[...] = hp[:, :_DHF]
    h2pB_ref[...] = hp[:, _DHF:]


_mid_call = pl.pallas_call(
    _mid_body,
    interpret=_IT,
    grid=(_G,),
    in_specs=[
        pl.BlockSpec((_R, _DH), lambda i: (i, 0)),
        pl.BlockSpec((1, _DH), lambda i: (0, 0)),
        pl.BlockSpec((1, _DH), lambda i: (0, 0)),
        pl.BlockSpec((1, _DH), lambda i: (0, 0)),
        pl.BlockSpec((1, _DH), lambda i: (0, 0)),
        pl.BlockSpec((_DH, _DH), lambda i: (0, 0)),
        pl.BlockSpec((_R, 1), lambda i: (i, 0)),
    ],
    out_specs=[
        pl.BlockSpec((_R, _DH), lambda i: (i, 0)),
        pl.BlockSpec((_R, _DHF), lambda i: (i, 0)),
        pl.BlockSpec((_R, _DHF), lambda i: (i, 0)),
    ],
    out_shape=(
        jax.ShapeDtypeStruct((_N, _DH), jnp.float32),
        jax.ShapeDtypeStruct((_N, _DHF), jnp.float32),
        jax.ShapeDtypeStruct((_N, _DHF), jnp.float32),
    ),
)


def _final_body(o_ref, psum_ref, psq_ref, g_ref, be_ref, batch_ref,
                fcw_ref, fcb_ref, out_ref, xmax_s, ssum_s, cnt_s):
    i = pl.program_id(0)
    h = _bn_relu(o_ref[...], psum_ref[...], psq_ref[...], g_ref[...], be_ref[...])
    bm = batch_ref[...]
    seg = lax.broadcasted_iota(jnp.int32, (_R, _NB), 1)
    mf = (bm == seg).astype(jnp.float32)
    ones = jnp.ones((_R, 1), jnp.float32)
    ss = lax.dot_general(mf, h, (((0,), (0,)), ((), ())),
                         preferred_element_type=jnp.float32)
    cn = lax.dot_general(mf, ones, (((0,), (0,)), ((), ())),
                         preferred_element_type=jnp.float32)

    @pl.when(i == 0)
    def _():
        xmax_s[...] = jnp.zeros_like(xmax_s)
        ssum_s[...] = jnp.zeros_like(ssum_s)
        cnt_s[...] = jnp.zeros_like(cnt_s)

    ssum_s[...] += ss
    cnt_s[...] += cn

    lo = jnp.min(bm)
    hi = jnp.max(bm)
    segid = lax.broadcasted_iota(jnp.int32, (_NB, 1), 0)

    def seg_step(sgi, carry):
        col = (bm == sgi).astype(jnp.float32)
        v = jnp.max(h * col, axis=0, keepdims=True)
        oh = (segid == sgi).astype(jnp.float32)
        xmax_s[...] = jnp.maximum(xmax_s[...], oh * v)
        return carry

    lax.fori_loop(lo, hi + 1, seg_step, 0)

    @pl.when(i == _G - 1)
    def _():
        smean = ssum_s[...] / jnp.maximum(cnt_s[...], 1.0)
        gemb = jnp.concatenate([xmax_s[...], smean], axis=1)
        out_ref[...] = (jnp.dot(gemb, fcw_ref[...],
                                preferred_element_type=jnp.float32)
                        + fcb_ref[...])


_final_call = pl.pallas_call(
    _final_body,
    interpret=_IT,
    grid=(_G,),
    in_specs=[
        pl.BlockSpec((_R, _DH), lambda i: (i, 0)),
        pl.BlockSpec((1, _DH), lambda i: (0, 0)),
        pl.BlockSpec((1, _DH), lambda i: (0, 0)),
        pl.BlockSpec((1, _DH), lambda i: (0, 0)),
        pl.BlockSpec((1, _DH), lambda i: (0, 0)),
        pl.BlockSpec((_R, 1), lambda i: (i, 0)),
        pl.BlockSpec((2 * _DH, 2), lambda i: (0, 0)),
        pl.BlockSpec((1, 2), lambda i: (0, 0)),
    ],
    out_specs=pl.BlockSpec((_NB, 2), lambda i: (0, 0)),
    out_shape=jax.ShapeDtypeStruct((_NB, 2), jnp.float32),
    scratch_shapes=[
        pltpu.VMEM((_NB, _DH), jnp.float32),
        pltpu.VMEM((_NB, _DH), jnp.float32),
        pltpu.VMEM((_NB, 1), jnp.float32),
    ],
)


def kernel(x, edge_index, batch, W1, b1, g1, be1, W2, b2, g2, be2, fcW, fcb):
    es = edge_index[0].reshape(_NS, _AGG_CH, _K)
    ed = edge_index[1].reshape(_NS, _AGG_CH, _K)
    edd = edge_index[1].reshape(_NC * _NS, _DEG_CH, _K)
    zrows = jnp.zeros((_N, _DHF), jnp.float32)
    zvec = jnp.zeros((_N,), jnp.float32)

    degP = _deg_call(edd, zvec).reshape(_NC, _N)
    degT = degP.T

    h1, hpA, hpB, dis = _lin1_call(x, W1, degT)
    aggA, aggB = _agg_call(hpA, hpB, es, ed, zrows)
    o1, ps1, pq1 = _stats_call(aggA, aggB, h1, dis, b1.reshape(1, -1))
    h2, h2pA, h2pB = _mid_call(o1, ps1, pq1, g1.reshape(1, -1),
                               be1.reshape(1, -1), W2, dis)
    agg2A, agg2B = _agg_call(h2pA, h2pB, es, ed, zrows)
    o2, ps2, pq2 = _stats_call(agg2A, agg2B, h2, dis, b2.reshape(1, -1))
    out = _final_call(o2, ps2, pq2, g2.reshape(1, -1), be2.reshape(1, -1),
                      batch.reshape(-1, 1), fcW, fcb.reshape(1, -1))
    return out


# speedup vs baseline: 22.8029x; 1.4976x over previous
import functools

_IT = False

import jax
import jax.numpy as jnp
from jax import lax
from jax.experimental import pallas as pl
from jax.experimental.pallas import tpu as pltpu
from jax.experimental.pallas import tpu_sc as plsc

_N = 10000
_E = 320000
_DIN = 128
_DH = 256
_DHF = 128
_NB = 64
_NC = 2
_NS = 16
_K = 125
_AGG_CH = _E // (_NS * _K)
_DEG_CH = _E // (_NC * _NS * _K)
_QCH = _AGG_CH // 4
_RPT = 624
_TAIL0 = _RPT * _NS
_TAILN = _N - _TAIL0

_R = 1000
_G = _N // _R

_mesh = plsc.VectorSubcoreMesh(core_axis_name="c", subcore_axis_name="s",
                               num_cores=_NC, num_subcores=_NS)


def _deg_body(e_dst, zvec, out, idx_v, ones_v, tmp_v, acc):
    c = lax.axis_index("c")
    s = lax.axis_index("s")
    wid = s * _NC + c
    pltpu.sync_copy(e_dst.at[wid], idx_v)
    for i in range(8):
        ones_v[pl.ds(i * 16, 16)] = jnp.ones((16,), jnp.float32)

    @pl.when(s == 0)
    def _():
        pltpu.sync_copy(zvec, tmp_v)
        pltpu.sync_copy(tmp_v, acc)

    plsc.subcore_barrier()

    def step(j, carry):
        pltpu.sync_copy(ones_v.at[pl.ds(0, _K)], acc.at[idx_v.at[j]], add=True)
        return carry

    lax.fori_loop(0, _DEG_CH, step, 0)
    plsc.subcore_barrier()

    @pl.when(s == 0)
    def _():
        pltpu.sync_copy(acc, tmp_v)
        pltpu.sync_copy(tmp_v, out.at[pl.ds(c * _N, _N)])


_deg_call = pl.kernel(
    _deg_body,
    out_type=jax.ShapeDtypeStruct((_NC * _N,), jnp.float32),
    mesh=_mesh,
    interpret=_IT,
    scratch_types=[
        pltpu.VMEM((_DEG_CH, _K), jnp.int32),
        pltpu.VMEM((128,), jnp.float32),
        pltpu.VMEM((_N,), jnp.float32),
        pltpu.VMEM_SHARED((_N,), jnp.float32),
    ],
)


def _agg_body(hpA, hpB, e_src, e_dst, zrows, outA, outB,
              sidx, didx, rbuf, acc, gsem):
    c = lax.axis_index("c")
    s = lax.axis_index("s")
    row0 = s * _RPT
    pltpu.sync_copy(zrows.at[pl.ds(row0, _RPT)], acc.at[pl.ds(row0, _RPT)])

    @pl.when(s == _NS - 1)
    def _():
        pltpu.sync_copy(zrows.at[pl.ds(_TAIL0, _TAILN)],
                        acc.at[pl.ds(_TAIL0, _TAILN)])

    plsc.subcore_barrier()

    def run(table):
        def step(j, carry):
            @pl.when(j + 1 < _QCH)
            def _():
                pltpu.async_copy(table.at[sidx.at[j + 1]],
                                 rbuf.at[(j + 1) % 2], gsem)
            pltpu.make_async_copy(table.at[sidx.at[j]],
                                  rbuf.at[j % 2], gsem).wait()
            pltpu.sync_copy(rbuf.at[j % 2], acc.at[didx.at[j]], add=True)
            return carry
        for p in range(4):
            pltpu.sync_copy(e_src.at[s, pl.ds(p * _QCH, _QCH)], sidx)
            pltpu.sync_copy(e_dst.at[s, pl.ds(p * _QCH, _QCH)], didx)
            pltpu.async_copy(table.at[sidx.at[0]], rbuf.at[0], gsem)
            lax.fori_loop(0, _QCH, step, 0)

    @pl.when(c == 0)
    def _():
        run(hpA)

    @pl.when(c == 1)
    def _():
        run(hpB)

    plsc.subcore_barrier()

    def copy_out(outref):
        pltpu.sync_copy(acc.at[pl.ds(row0, _RPT)], outref.at[pl.ds(row0, _RPT)])

        @pl.when(s == _NS - 1)
        def _():
            pltpu.sync_copy(acc.at[pl.ds(_TAIL0, _TAILN)],
                            outref.at[pl.ds(_TAIL0, _TAILN)])

    @pl.when(c == 0)
    def _():
        copy_out(outA)

    @pl.when(c == 1)
    def _():
        copy_out(outB)


_agg_call = pl.kernel(
    _agg_body,
    out_type=(jax.ShapeDtypeStruct((_N, _DHF), jnp.float32),
              jax.ShapeDtypeStruct((_N, _DHF), jnp.float32)),
    mesh=_mesh,
    interpret=_IT,
    scratch_types=[
        pltpu.VMEM((_QCH, _K), jnp.int32),
        pltpu.VMEM((_QCH, _K), jnp.int32),
        pltpu.VMEM((2, _K, _DHF), jnp.float32),
        pltpu.VMEM_SHARED((_N, _DHF), jnp.float32),
        pltpu.SemaphoreType.DMA,
    ],
)


def _lin1_body(x_ref, w1_ref, degT_ref, h1_ref, hpA_ref, hpB_ref, dis_ref):
    deg = degT_ref[:, 0:1] + degT_ref[:, 1:2] + 1.0
    dis = lax.rsqrt(deg)
    h1 = jnp.dot(x_ref[...], w1_ref[...], preferred_element_type=jnp.float32)
    hp = h1 * dis
    h1_ref[...] = h1
    hpA_ref[...] = hp[:, :_DHF]
    hpB_ref[...] = hp[:, _DHF:]
    dis_ref[...] = dis


_lin1_call = pl.pallas_call(
    _lin1_body,
    interpret=_IT,
    grid=(_G,),
    in_specs=[
        pl.BlockSpec((_R, _DIN), lambda i: (i, 0)),
        pl.BlockSpec((_DIN, _DH), lambda i: (0, 0)),
        pl.BlockSpec((_R, 2), lambda i: (i, 0)),
    ],
    out_specs=[
        pl.BlockSpec((_R, _DH), lambda i: (i, 0)),
        pl.BlockSpec((_R, _DHF), lambda i: (i, 0)),
        pl.BlockSpec((_R, _DHF), lambda i: (i, 0)),
        pl.BlockSpec((_R, 1), lambda i: (i, 0)),
    ],
    out_shape=(
        jax.ShapeDtypeStruct((_N, _DH), jnp.float32),
        jax.ShapeDtypeStruct((_N, _DHF), jnp.float32),
        jax.ShapeDtypeStruct((_N, _DHF), jnp.float32),
        jax.ShapeDtypeStruct((_N, 1), jnp.float32),
    ),
)


def _stats_body(aggA_ref, aggB_ref, hself_ref, dis_ref, b_ref,
                o_ref, psum_ref, psq_ref):
    i = pl.program_id(0)
    dis = dis_ref[...]
    agg = jnp.concatenate([aggA_ref[...], aggB_ref[...]], axis=1)
    o = dis * agg + (dis * dis) * hself_ref[...] + b_ref[...]
    o_ref[...] = o

    @pl.when(i == 0)
    def _():
        psum_ref[...] = jnp.zeros_like(psum_ref)
        psq_ref[...] = jnp.zeros_like(psq_ref)

    psum_ref[...] += jnp.sum(o, axis=0, keepdims=True)
    psq_ref[...] += jnp.sum(o * o, axis=0, keepdims=True)


_stats_call = pl.pallas_call(
    _stats_body,
    interpret=_IT,
    grid=(_G,),
    in_specs=[
        pl.BlockSpec((_R, _DHF), lambda i: (i, 0)),
        pl.BlockSpec((_R, _DHF), lambda i: (i, 0)),
        pl.BlockSpec((_R, _DH), lambda i: (i, 0)),
        pl.BlockSpec((_R, 1), lambda i: (i, 0)),
        pl.BlockSpec((1, _DH), lambda i: (0, 0)),
    ],
    out_specs=[
        pl.BlockSpec((_R, _DH), lambda i: (i, 0)),
        pl.BlockSpec((1, _DH), lambda i: (0, 0)),
        pl.BlockSpec((1, _DH), lambda i: (0, 0)),
    ],
    out_shape=(
        jax.ShapeDtypeStruct((_N, _DH), jnp.float32),
        jax.ShapeDtypeStruct((1, _DH), jnp.float32),
        jax.ShapeDtypeStruct((1, _DH), jnp.float32),
    ),
)


def _bn_relu(o, psum, psq, g, be):
    mu = psum * (1.0 / _N)
    var = psq * (1.0 / _N) - mu * mu
    return jnp.maximum(g * (o - mu) * lax.rsqrt(var + 1e-5) + be, 0.0)


def _mid_body(o_ref, psum_ref, psq_ref, g_ref, be_ref, w2_ref, dis_ref,
              h2_ref, h2pA_ref, h2pB_ref):
    h = _bn_relu(o_ref[...], psum_ref[...], psq_ref[...], g_ref[...], be_ref[...])
    h2 = jnp.dot(h, w2_ref[...], preferred_element_type=jnp.float32)
    h2_ref[...] = h2
    hp = h2 * dis_ref[...]
    h2pA_ref[...] = hp[:, :_DHF]
    h2pB_ref[...] = hp[:, _DHF:]


_mid_call = pl.pallas_call(
    _mid_body,
    interpret=_IT,
    grid=(_G,),
    in_specs=[
        pl.BlockSpec((_R, _DH), lambda i: (i, 0)),
        pl.BlockSpec((1, _DH), lambda i: (0, 0)),
        pl.BlockSpec((1, _DH), lambda i: (0, 0)),
        pl.BlockSpec((1, _DH), lambda i: (0, 0)),
        pl.BlockSpec((1, _DH), lambda i: (0, 0)),
        pl.BlockSpec((_DH, _DH), lambda i: (0, 0)),
        pl.BlockSpec((_R, 1), lambda i: (i, 0)),
    ],
    out_specs=[
        pl.BlockSpec((_R, _DH), lambda i: (i, 0)),
        pl.BlockSpec((_R, _DHF), lambda i: (i, 0)),
        pl.BlockSpec((_R, _DHF), lambda i: (i, 0)),
    ],
    out_shape=(
        jax.ShapeDtypeStruct((_N, _DH), jnp.float32),
        jax.ShapeDtypeStruct((_N, _DHF), jnp.float32),
        jax.ShapeDtypeStruct((_N, _DHF), jnp.float32),
    ),
)


def _final_body(o_ref, psum_ref, psq_ref, g_ref, be_ref, batch_ref,
                fcw_ref, fcb_ref, out_ref, xmax_s, ssum_s, cnt_s):
    i = pl.program_id(0)
    h = _bn_relu(o_ref[...], psum_ref[...], psq_ref[...], g_ref[...], be_ref[...])
    bm = batch_ref[...]
    seg = lax.broadcasted_iota(jnp.int32, (_R, _NB), 1)
    mf = (bm == seg).astype(jnp.float32)
    ones = jnp.ones((_R, 1), jnp.float32)
    ss = lax.dot_general(mf, h, (((0,), (0,)), ((), ())),
                         preferred_element_type=jnp.float32)
    cn = lax.dot_general(mf, ones, (((0,), (0,)), ((), ())),
                         preferred_element_type=jnp.float32)

    @pl.when(i == 0)
    def _():
        xmax_s[...] = jnp.zeros_like(xmax_s)
        ssum_s[...] = jnp.zeros_like(ssum_s)
        cnt_s[...] = jnp.zeros_like(cnt_s)

    ssum_s[...] += ss
    cnt_s[...] += cn

    lo = jnp.min(bm)
    hi = jnp.max(bm)
    segid = lax.broadcasted_iota(jnp.int32, (_NB, 1), 0)

    def seg_step(sgi, carry):
        col = (bm == sgi).astype(jnp.float32)
        v = jnp.max(h * col, axis=0, keepdims=True)
        oh = (segid == sgi).astype(jnp.float32)
        xmax_s[...] = jnp.maximum(xmax_s[...], oh * v)
        return carry

    lax.fori_loop(lo, hi + 1, seg_step, 0)

    @pl.when(i == _G - 1)
    def _():
        smean = ssum_s[...] / jnp.maximum(cnt_s[...], 1.0)
        gemb = jnp.concatenate([xmax_s[...], smean], axis=1)
        out_ref[...] = (jnp.dot(gemb, fcw_ref[...],
                                preferred_element_type=jnp.float32)
                        + fcb_ref[...])


_final_call = pl.pallas_call(
    _final_body,
    interpret=_IT,
    grid=(_G,),
    in_specs=[
        pl.BlockSpec((_R, _DH), lambda i: (i, 0)),
        pl.BlockSpec((1, _DH), lambda i: (0, 0)),
        pl.BlockSpec((1, _DH), lambda i: (0, 0)),
        pl.BlockSpec((1, _DH), lambda i: (0, 0)),
        pl.BlockSpec((1, _DH), lambda i: (0, 0)),
        pl.BlockSpec((_R, 1), lambda i: (i, 0)),
        pl.BlockSpec((2 * _DH, 2), lambda i: (0, 0)),
        pl.BlockSpec((1, 2), lambda i: (0, 0)),
    ],
    out_specs=pl.BlockSpec((_NB, 2), lambda i: (0, 0)),
    out_shape=jax.ShapeDtypeStruct((_NB, 2), jnp.float32),
    scratch_shapes=[
        pltpu.VMEM((_NB, _DH), jnp.float32),
        pltpu.VMEM((_NB, _DH), jnp.float32),
        pltpu.VMEM((_NB, 1), jnp.float32),
    ],
)


def kernel(x, edge_index, batch, W1, b1, g1, be1, W2, b2, g2, be2, fcW, fcb):
    es = edge_index[0].reshape(_NS, _AGG_CH, _K)
    ed = edge_index[1].reshape(_NS, _AGG_CH, _K)
    edd = edge_index[1].reshape(_NC * _NS, _DEG_CH, _K)
    zrows = jnp.zeros((_N, _DHF), jnp.float32)
    zvec = jnp.zeros((_N,), jnp.float32)

    degP = _deg_call(edd, zvec).reshape(_NC, _N)
    degT = degP.T

    h1, hpA, hpB, dis = _lin1_call(x, W1, degT)
    aggA, aggB = _agg_call(hpA, hpB, es, ed, zrows)
    o1, ps1, pq1 = _stats_call(aggA, aggB, h1, dis, b1.reshape(1, -1))
    h2, h2pA, h2pB = _mid_call(o1, ps1, pq1, g1.reshape(1, -1),
                               be1.reshape(1, -1), W2, dis)
    agg2A, agg2B = _agg_call(h2pA, h2pB, es, ed, zrows)
    o2, ps2, pq2 = _stats_call(agg2A, agg2B, h2, dis, b2.reshape(1, -1))
    out = _final_call(o2, ps2, pq2, g2.reshape(1, -1), be2.reshape(1, -1),
                      batch.reshape(-1, 1), fcW, fcb.reshape(1, -1))
    return out


# speedup vs baseline: 23.6472x; 1.0370x over previous
import functools

_IT = False

import jax
import jax.numpy as jnp
from jax import lax
from jax.experimental import pallas as pl
from jax.experimental.pallas import tpu as pltpu
from jax.experimental.pallas import tpu_sc as plsc

_N = 10000
_E = 320000
_DIN = 128
_DH = 256
_DHF = 128
_NB = 64
_NC = 2
_NS = 16
_K = 100
_AGG_CH = _E // (_NS * _K)
_DEG_CH = _E // (_NC * _NS * _K)
_QCH = 40
_NPH = _AGG_CH // _QCH
_RPT = 624
_TAIL0 = _RPT * _NS
_TAILN = _N - _TAIL0

_R = 1000
_G = _N // _R

_mesh = plsc.VectorSubcoreMesh(core_axis_name="c", subcore_axis_name="s",
                               num_cores=_NC, num_subcores=_NS)


def _deg_body(e_dst, zvec, out, idx_v, ones_v, tmp_v, acc):
    c = lax.axis_index("c")
    s = lax.axis_index("s")
    wid = s * _NC + c
    pltpu.sync_copy(e_dst.at[wid], idx_v)
    for i in range(8):
        ones_v[pl.ds(i * 16, 16)] = jnp.ones((16,), jnp.float32)

    @pl.when(s == 0)
    def _():
        pltpu.sync_copy(zvec, tmp_v)
        pltpu.sync_copy(tmp_v, acc)

    plsc.subcore_barrier()

    def step(j, carry):
        pltpu.sync_copy(ones_v.at[pl.ds(0, _K)], acc.at[idx_v.at[j]], add=True)
        return carry

    lax.fori_loop(0, _DEG_CH, step, 0)
    plsc.subcore_barrier()

    @pl.when(s == 0)
    def _():
        pltpu.sync_copy(acc, tmp_v)
        pltpu.sync_copy(tmp_v, out.at[pl.ds(c * _N, _N)])


_deg_call = pl.kernel(
    _deg_body,
    out_type=jax.ShapeDtypeStruct((_NC * _N,), jnp.float32),
    mesh=_mesh,
    interpret=_IT,
    scratch_types=[
        pltpu.VMEM((_DEG_CH, _K), jnp.int32),
        pltpu.VMEM((128,), jnp.float32),
        pltpu.VMEM((_N,), jnp.float32),
        pltpu.VMEM_SHARED((_N,), jnp.float32),
    ],
)


def _agg_body(hpA, hpB, e_src, e_dst, zrows, outA, outB,
              sidx, didx, rbuf, acc, gsem, ssem):
    c = lax.axis_index("c")
    s = lax.axis_index("s")
    row0 = s * _RPT
    pltpu.sync_copy(zrows.at[pl.ds(row0, _RPT)], acc.at[pl.ds(row0, _RPT)])

    @pl.when(s == _NS - 1)
    def _():
        pltpu.sync_copy(zrows.at[pl.ds(_TAIL0, _TAILN)],
                        acc.at[pl.ds(_TAIL0, _TAILN)])

    plsc.subcore_barrier()

    def run(table):
        def gather(j):
            pltpu.async_copy(table.at[sidx.at[j]], rbuf.at[j % 3],
                             gsem.at[j % 3])

        def step(j, carry):
            pltpu.make_async_copy(table.at[sidx.at[j]], rbuf.at[j % 3],
                                  gsem.at[j % 3]).wait()
            pltpu.async_copy(rbuf.at[j % 3], acc.at[didx.at[j]],
                             ssem.at[j % 3], add=True)

            @pl.when(j >= 1)
            def _():
                pltpu.make_async_copy(rbuf.at[(j - 1) % 3],
                                      acc.at[didx.at[j - 1]],
                                      ssem.at[(j - 1) % 3]).wait()

            @pl.when(j + 2 < _QCH)
            def _():
                gather(j + 2)

            return carry

        for p in range(_NPH):
            pltpu.sync_copy(e_src.at[s, pl.ds(p * _QCH, _QCH)], sidx)
            pltpu.sync_copy(e_dst.at[s, pl.ds(p * _QCH, _QCH)], didx)
            gather(0)
            gather(1)
            lax.fori_loop(0, _QCH, step, 0)
            pltpu.make_async_copy(rbuf.at[(_QCH - 1) % 3],
                                  acc.at[didx.at[_QCH - 1]],
                                  ssem.at[(_QCH - 1) % 3]).wait()

    @pl.when(c == 0)
    def _():
        run(hpA)

    @pl.when(c == 1)
    def _():
        run(hpB)

    plsc.subcore_barrier()

    def copy_out(outref):
        pltpu.sync_copy(acc.at[pl.ds(row0, _RPT)], outref.at[pl.ds(row0, _RPT)])

        @pl.when(s == _NS - 1)
        def _():
            pltpu.sync_copy(acc.at[pl.ds(_TAIL0, _TAILN)],
                            outref.at[pl.ds(_TAIL0, _TAILN)])

    @pl.when(c == 0)
    def _():
        copy_out(outA)

    @pl.when(c == 1)
    def _():
        copy_out(outB)


_agg_call = pl.kernel(
    _agg_body,
    out_type=(jax.ShapeDtypeStruct((_N, _DHF), jnp.float32),
              jax.ShapeDtypeStruct((_N, _DHF), jnp.float32)),
    mesh=_mesh,
    interpret=_IT,
    scratch_types=[
        pltpu.VMEM((_QCH, _K), jnp.int32),
        pltpu.VMEM((_QCH, _K), jnp.int32),
        pltpu.VMEM((3, _K, _DHF), jnp.float32),
        pltpu.VMEM_SHARED((_N, _DHF), jnp.float32),
        pltpu.SemaphoreType.DMA((3,)),
        pltpu.SemaphoreType.DMA((3,)),
    ],
)


def _lin1_body(x_ref, w1_ref, degT_ref, h1_ref, hpA_ref, hpB_ref, dis_ref):
    deg = degT_ref[:, 0:1] + degT_ref[:, 1:2] + 1.0
    dis = lax.rsqrt(deg)
    h1 = jnp.dot(x_ref[...], w1_ref[...], preferred_element_type=jnp.float32)
    hp = h1 * dis
    h1_ref[...] = h1
    hpA_ref[...] = hp[:, :_DHF]
    hpB_ref[...] = hp[:, _DHF:]
    dis_ref[...] = dis


_lin1_call = pl.pallas_call(
    _lin1_body,
    interpret=_IT,
    grid=(_G,),
    in_specs=[
        pl.BlockSpec((_R, _DIN), lambda i: (i, 0)),
        pl.BlockSpec((_DIN, _DH), lambda i: (0, 0)),
        pl.BlockSpec((_R, 2), lambda i: (i, 0)),
    ],
    out_specs=[
        pl.BlockSpec((_R, _DH), lambda i: (i, 0)),
        pl.BlockSpec((_R, _DHF), lambda i: (i, 0)),
        pl.BlockSpec((_R, _DHF), lambda i: (i, 0)),
        pl.BlockSpec((_R, 1), lambda i: (i, 0)),
    ],
    out_shape=(
        jax.ShapeDtypeStruct((_N, _DH), jnp.float32),
        jax.ShapeDtypeStruct((_N, _DHF), jnp.float32),
        jax.ShapeDtypeStruct((_N, _DHF), jnp.float32),
        jax.ShapeDtypeStruct((_N, 1), jnp.float32),
    ),
)


def _stats_body(aggA_ref, aggB_ref, hself_ref, dis_ref, b_ref,
                o_ref, psum_ref, psq_ref):
    i = pl.program_id(0)
    dis = dis_ref[...]
    agg = jnp.concatenate([aggA_ref[...], aggB_ref[...]], axis=1)
    o = dis * agg + (dis * dis) * hself_ref[...] + b_ref[...]
    o_ref[...] = o

    @pl.when(i == 0)
    def _():
        psum_ref[...] = jnp.zeros_like(psum_ref)
        psq_ref[...] = jnp.zeros_like(psq_ref)

    psum_ref[...] += jnp.sum(o, axis=0, keepdims=True)
    psq_ref[...] += jnp.sum(o * o, axis=0, keepdims=True)


_stats_call = pl.pallas_call(
    _stats_body,
    interpret=_IT,
    grid=(_G,),
    in_specs=[
        pl.BlockSpec((_R, _DHF), lambda i: (i, 0)),
        pl.BlockSpec((_R, _DHF), lambda i: (i, 0)),
        pl.BlockSpec((_R, _DH), lambda i: (i, 0)),
        pl.BlockSpec((_R, 1), lambda i: (i, 0)),
        pl.BlockSpec((1, _DH), lambda i: (0, 0)),
    ],
    out_specs=[
        pl.BlockSpec((_R, _DH), lambda i: (i, 0)),
        pl.BlockSpec((1, _DH), lambda i: (0, 0)),
        pl.BlockSpec((1, _DH), lambda i: (0, 0)),
    ],
    out_shape=(
        jax.ShapeDtypeStruct((_N, _DH), jnp.float32),
        jax.ShapeDtypeStruct((1, _DH), jnp.float32),
        jax.ShapeDtypeStruct((1, _DH), jnp.float32),
    ),
)


def _bn_relu(o, psum, psq, g, be):
    mu = psum * (1.0 / _N)
    var = psq * (1.0 / _N) - mu * mu
    return jnp.maximum(g * (o - mu) * lax.rsqrt(var + 1e-5) + be, 0.0)


def _mid_body(o_ref, psum_ref, psq_ref, g_ref, be_ref, w2_ref, dis_ref,
              h2_ref, h2pA_ref, h2pB_ref):
    h = _bn_relu(o_ref[...], psum_ref[...], psq_ref[...], g_ref[...], be_ref[...])
    h2 = jnp.dot(h, w2_ref[...], preferred_element_type=jnp.float32)
    h2_ref[...] = h2
    hp = h2 * dis_ref[...]
    h2pA_ref[...] = hp[:, :_DHF]
    h2pB_ref[...] = hp[:, _DHF:]


_mid_call = pl.pallas_call(
    _mid_body,
    interpret=_IT,
    grid=(_G,),
    in_specs=[
        pl.BlockSpec((_R, _DH), lambda i: (i, 0)),
        pl.BlockSpec((1, _DH), lambda i: (0, 0)),
        pl.BlockSpec((1, _DH), lambda i: (0, 0)),
        pl.BlockSpec((1, _DH), lambda i: (0, 0)),
        pl.BlockSpec((1, _DH), lambda i: (0, 0)),
        pl.BlockSpec((_DH, _DH), lambda i: (0, 0)),
        pl.BlockSpec((_R, 1), lambda i: (i, 0)),
    ],
    out_specs=[
        pl.BlockSpec((_R, _DH), lambda i: (i, 0)),
        pl.BlockSpec((_R, _DHF), lambda i: (i, 0)),
        pl.BlockSpec((_R, _DHF), lambda i: (i, 0)),
    ],
    out_shape=(
        jax.ShapeDtypeStruct((_N, _DH), jnp.float32),
        jax.ShapeDtypeStruct((_N, _DHF), jnp.float32),
        jax.ShapeDtypeStruct((_N, _DHF), jnp.float32),
    ),
)


def _final_body(o_ref, psum_ref, psq_ref, g_ref, be_ref, batch_ref,
                fcw_ref, fcb_ref, out_ref, xmax_s, ssum_s, cnt_s):
    i = pl.program_id(0)
    h = _bn_relu(o_ref[...], psum_ref[...], psq_ref[...], g_ref[...], be_ref[...])
    bm = batch_ref[...]
    seg = lax.broadcasted_iota(jnp.int32, (_R, _NB), 1)
    mf = (bm == seg).astype(jnp.float32)
    ones = jnp.ones((_R, 1), jnp.float32)
    ss = lax.dot_general(mf, h, (((0,), (0,)), ((), ())),
                         preferred_element_type=jnp.float32)
    cn = lax.dot_general(mf, ones, (((0,), (0,)), ((), ())),
                         preferred_element_type=jnp.float32)

    @pl.when(i == 0)
    def _():
        xmax_s[...] = jnp.zeros_like(xmax_s)
        ssum_s[...] = jnp.zeros_like(ssum_s)
        cnt_s[...] = jnp.zeros_like(cnt_s)

    ssum_s[...] += ss
    cnt_s[...] += cn

    lo = jnp.min(bm)
    hi = jnp.max(bm)
    segid = lax.broadcasted_iota(jnp.int32, (_NB, 1), 0)

    def seg_step(sgi, carry):
        col = (bm == sgi).astype(jnp.float32)
        v = jnp.max(h * col, axis=0, keepdims=True)
        oh = (segid == sgi).astype(jnp.float32)
        xmax_s[...] = jnp.maximum(xmax_s[...], oh * v)
        return carry

    lax.fori_loop(lo, hi + 1, seg_step, 0)

    @pl.when(i == _G - 1)
    def _():
        smean = ssum_s[...] / jnp.maximum(cnt_s[...], 1.0)
        gemb = jnp.concatenate([xmax_s[...], smean], axis=1)
        out_ref[...] = (jnp.dot(gemb, fcw_ref[...],
                                preferred_element_type=jnp.float32)
                        + fcb_ref[...])


_final_call = pl.pallas_call(
    _final_body,
    interpret=_IT,
    grid=(_G,),
    in_specs=[
        pl.BlockSpec((_R, _DH), lambda i: (i, 0)),
        pl.BlockSpec((1, _DH), lambda i: (0, 0)),
        pl.BlockSpec((1, _DH), lambda i: (0, 0)),
        pl.BlockSpec((1, _DH), lambda i: (0, 0)),
        pl.BlockSpec((1, _DH), lambda i: (0, 0)),
        pl.BlockSpec((_R, 1), lambda i: (i, 0)),
        pl.BlockSpec((2 * _DH, 2), lambda i: (0, 0)),
        pl.BlockSpec((1, 2), lambda i: (0, 0)),
    ],
    out_specs=pl.BlockSpec((_NB, 2), lambda i: (0, 0)),
    out_shape=jax.ShapeDtypeStruct((_NB, 2), jnp.float32),
    scratch_shapes=[
        pltpu.VMEM((_NB, _DH), jnp.float32),
        pltpu.VMEM((_NB, _DH), jnp.float32),
        pltpu.VMEM((_NB, 1), jnp.float32),
    ],
)


def kernel(x, edge_index, batch, W1, b1, g1, be1, W2, b2, g2, be2, fcW, fcb):
    es = edge_index[0].reshape(_NS, _AGG_CH, _K)
    ed = edge_index[1].reshape(_NS, _AGG_CH, _K)
    edd = edge_index[1].reshape(_NC * _NS, _DEG_CH, _K)
    zrows = jnp.zeros((_N, _DHF), jnp.float32)
    zvec = jnp.zeros((_N,), jnp.float32)

    degP = _deg_call(edd, zvec).reshape(_NC, _N)
    degT = degP.T

    h1, hpA, hpB, dis = _lin1_call(x, W1, degT)
    aggA, aggB = _agg_call(hpA, hpB, es, ed, zrows)
    o1, ps1, pq1 = _stats_call(aggA, aggB, h1, dis, b1.reshape(1, -1))
    h2, h2pA, h2pB = _mid_call(o1, ps1, pq1, g1.reshape(1, -1),
                               be1.reshape(1, -1), W2, dis)
    agg2A, agg2B = _agg_call(h2pA, h2pB, es, ed, zrows)
    o2, ps2, pq2 = _stats_call(agg2A, agg2B, h2, dis, b2.reshape(1, -1))
    out = _final_call(o2, ps2, pq2, g2.reshape(1, -1), be2.reshape(1, -1),
                      batch.reshape(-1, 1), fcW, fcb.reshape(1, -1))
    return out


# speedup vs baseline: 23.9844x; 1.0143x over previous
import functools

_IT = False

import jax
import jax.numpy as jnp
from jax import lax
from jax.experimental import pallas as pl
from jax.experimental.pallas import tpu as pltpu
from jax.experimental.pallas import tpu_sc as plsc

_N = 10000
_E = 320000
_DIN = 128
_DH = 256
_DHF = 128
_NB = 64
_NC = 2
_NS = 16
_K = 100
_AGG_CH = _E // (_NS * _K)
_DEG_CH = _E // (_NC * _NS * _K)
_QCH = 40
_NPH = _AGG_CH // _QCH
_RPT = 624
_TAIL0 = _RPT * _NS
_TAILN = _N - _TAIL0

_R = 1000
_G = _N // _R

_mesh = plsc.VectorSubcoreMesh(core_axis_name="c", subcore_axis_name="s",
                               num_cores=_NC, num_subcores=_NS)


def _deg_body(e_dst, zvec, out, idx_v, ones_v, tmp_v, acc):
    c = lax.axis_index("c")
    s = lax.axis_index("s")
    wid = s * _NC + c
    pltpu.sync_copy(e_dst.at[wid], idx_v)
    for i in range(8):
        ones_v[pl.ds(i * 16, 16)] = jnp.ones((16,), jnp.float32)

    @pl.when(s == 0)
    def _():
        pltpu.sync_copy(zvec, tmp_v)
        pltpu.sync_copy(tmp_v, acc)

    plsc.subcore_barrier()

    def step(j, carry):
        pltpu.sync_copy(ones_v.at[pl.ds(0, _K)], acc.at[idx_v.at[j]], add=True)
        return carry

    lax.fori_loop(0, _DEG_CH, step, 0)
    plsc.subcore_barrier()

    @pl.when(s == 0)
    def _():
        pltpu.sync_copy(acc, tmp_v)
        pltpu.sync_copy(tmp_v, out.at[pl.ds(c * _N, _N)])


_deg_call = pl.kernel(
    _deg_body,
    out_type=jax.ShapeDtypeStruct((_NC * _N,), jnp.float32),
    mesh=_mesh,
    interpret=_IT,
    scratch_types=[
        pltpu.VMEM((_DEG_CH, _K), jnp.int32),
        pltpu.VMEM((128,), jnp.float32),
        pltpu.VMEM((_N,), jnp.float32),
        pltpu.VMEM_SHARED((_N,), jnp.float32),
    ],
)


def _agg_body(hpA, hpB, e_src, e_dst, zrows, outA, outB,
              sidx, didx, rbuf, acc, gsem, ssem):
    c = lax.axis_index("c")
    s = lax.axis_index("s")
    row0 = s * _RPT
    pltpu.sync_copy(zrows.at[pl.ds(row0, _RPT)], acc.at[pl.ds(row0, _RPT)])

    @pl.when(s == _NS - 1)
    def _():
        pltpu.sync_copy(zrows.at[pl.ds(_TAIL0, _TAILN)],
                        acc.at[pl.ds(_TAIL0, _TAILN)])

    plsc.subcore_barrier()

    def run(table):
        def gather(j):
            pltpu.async_copy(table.at[sidx.at[j]], rbuf.at[j % 3],
                             gsem.at[j % 3])

        def step(j, carry):
            pltpu.make_async_copy(table.at[sidx.at[j]], rbuf.at[j % 3],
                                  gsem.at[j % 3]).wait()
            pltpu.async_copy(rbuf.at[j % 3], acc.at[didx.at[j]],
                             ssem.at[j % 3], add=True)

            @pl.when(j >= 1)
            def _():
                pltpu.make_async_copy(rbuf.at[(j - 1) % 3],
                                      acc.at[didx.at[j - 1]],
                                      ssem.at[(j - 1) % 3]).wait()

            @pl.when(j + 2 < _QCH)
            def _():
                gather(j + 2)

            return carry

        for p in range(_NPH):
            pltpu.sync_copy(e_src.at[s, pl.ds(p * _QCH, _QCH)], sidx)
            pltpu.sync_copy(e_dst.at[s, pl.ds(p * _QCH, _QCH)], didx)
            gather(0)
            gather(1)
            lax.fori_loop(0, _QCH, step, 0)
            pltpu.make_async_copy(rbuf.at[(_QCH - 1) % 3],
                                  acc.at[didx.at[_QCH - 1]],
                                  ssem.at[(_QCH - 1) % 3]).wait()

    @pl.when(c == 0)
    def _():
        run(hpA)

    @pl.when(c == 1)
    def _():
        run(hpB)

    plsc.subcore_barrier()

    def copy_out(outref):
        pltpu.sync_copy(acc.at[pl.ds(row0, _RPT)], outref.at[pl.ds(row0, _RPT)])

        @pl.when(s == _NS - 1)
        def _():
            pltpu.sync_copy(acc.at[pl.ds(_TAIL0, _TAILN)],
                            outref.at[pl.ds(_TAIL0, _TAILN)])

    @pl.when(c == 0)
    def _():
        copy_out(outA)

    @pl.when(c == 1)
    def _():
        copy_out(outB)


_agg_call = pl.kernel(
    _agg_body,
    out_type=(jax.ShapeDtypeStruct((_N, _DHF), jnp.float32),
              jax.ShapeDtypeStruct((_N, _DHF), jnp.float32)),
    mesh=_mesh,
    interpret=_IT,
    scratch_types=[
        pltpu.VMEM((_QCH, _K), jnp.int32),
        pltpu.VMEM((_QCH, _K), jnp.int32),
        pltpu.VMEM((3, _K, _DHF), jnp.float32),
        pltpu.VMEM_SHARED((_N, _DHF), jnp.float32),
        pltpu.SemaphoreType.DMA((3,)),
        pltpu.SemaphoreType.DMA((3,)),
    ],
)


def _lin1_body(x_ref, w1_ref, degT_ref, h1_ref, hpA_ref, hpB_ref, dis_ref):
    deg = degT_ref[:, 0:1] + degT_ref[:, 1:2] + 1.0
    dis = lax.rsqrt(deg)
    h1 = jnp.dot(x_ref[...], w1_ref[...], preferred_element_type=jnp.float32)
    hp = h1 * dis
    h1_ref[...] = h1
    hpA_ref[...] = hp[:, :_DHF]
    hpB_ref[...] = hp[:, _DHF:]
    dis_ref[...] = dis


_lin1_call = pl.pallas_call(
    _lin1_body,
    interpret=_IT,
    grid=(_G,),
    in_specs=[
        pl.BlockSpec((_R, _DIN), lambda i: (i, 0)),
        pl.BlockSpec((_DIN, _DH), lambda i: (0, 0)),
        pl.BlockSpec((_R, 2), lambda i: (i, 0)),
    ],
    out_specs=[
        pl.BlockSpec((_R, _DH), lambda i: (i, 0)),
        pl.BlockSpec((_R, _DHF), lambda i: (i, 0)),
        pl.BlockSpec((_R, _DHF), lambda i: (i, 0)),
        pl.BlockSpec((_R, 1), lambda i: (i, 0)),
    ],
    out_shape=(
        jax.ShapeDtypeStruct((_N, _DH), jnp.float32),
        jax.ShapeDtypeStruct((_N, _DHF), jnp.float32),
        jax.ShapeDtypeStruct((_N, _DHF), jnp.float32),
        jax.ShapeDtypeStruct((_N, 1), jnp.float32),
    ),
)


def _stats_body(aggA_ref, aggB_ref, hself_ref, dis_ref, b_ref,
                o_ref, psum_ref, psq_ref):
    i = pl.program_id(0)
    dis = dis_ref[...]
    agg = jnp.concatenate([aggA_ref[...], aggB_ref[...]], axis=1)
    o = dis * agg + (dis * dis) * hself_ref[...] + b_ref[...]
    o_ref[...] = o

    @pl.when(i == 0)
    def _():
        psum_ref[...] = jnp.zeros_like(psum_ref)
        psq_ref[...] = jnp.zeros_like(psq_ref)

    psum_ref[...] += jnp.sum(o, axis=0, keepdims=True)
    psq_ref[...] += jnp.sum(o * o, axis=0, keepdims=True)


_stats_call = pl.pallas_call(
    _stats_body,
    interpret=_IT,
    grid=(_G,),
    in_specs=[
        pl.BlockSpec((_R, _DHF), lambda i: (i, 0)),
        pl.BlockSpec((_R, _DHF), lambda i: (i, 0)),
        pl.BlockSpec((_R, _DH), lambda i: (i, 0)),
        pl.BlockSpec((_R, 1), lambda i: (i, 0)),
        pl.BlockSpec((1, _DH), lambda i: (0, 0)),
    ],
    out_specs=[
        pl.BlockSpec((_R, _DH), lambda i: (i, 0)),
        pl.BlockSpec((1, _DH), lambda i: (0, 0)),
        pl.BlockSpec((1, _DH), lambda i: (0, 0)),
    ],
    out_shape=(
        jax.ShapeDtypeStruct((_N, _DH), jnp.float32),
        jax.ShapeDtypeStruct((1, _DH), jnp.float32),
        jax.ShapeDtypeStruct((1, _DH), jnp.float32),
    ),
)


def _bn_relu(o, psum, psq, g, be):
    mu = psum * (1.0 / _N)
    var = psq * (1.0 / _N) - mu * mu
    return jnp.maximum(g * (o - mu) * lax.rsqrt(var + 1e-5) + be, 0.0)


def _mid_body(o_ref, psum_ref, psq_ref, g_ref, be_ref, w2_ref, dis_ref,
              h2_ref, h2pA_ref, h2pB_ref):
    h = _bn_relu(o_ref[...], psum_ref[...], psq_ref[...], g_ref[...], be_ref[...])
    h2 = jnp.dot(h, w2_ref[...], preferred_element_type=jnp.float32)
    h2_ref[...] = h2
    hp = h2 * dis_ref[...]
    h2pA_ref[...] = hp[:, :_DHF]
    h2pB_ref[...] = hp[:, _DHF:]


_mid_call = pl.pallas_call(
    _mid_body,
    interpret=_IT,
    grid=(_G,),
    in_specs=[
        pl.BlockSpec((_R, _DH), lambda i: (i, 0)),
        pl.BlockSpec((1, _DH), lambda i: (0, 0)),
        pl.BlockSpec((1, _DH), lambda i: (0, 0)),
        pl.BlockSpec((1, _DH), lambda i: (0, 0)),
        pl.BlockSpec((1, _DH), lambda i: (0, 0)),
        pl.BlockSpec((_DH, _DH), lambda i: (0, 0)),
        pl.BlockSpec((_R, 1), lambda i: (i, 0)),
    ],
    out_specs=[
        pl.BlockSpec((_R, _DH), lambda i: (i, 0)),
        pl.BlockSpec((_R, _DHF), lambda i: (i, 0)),
        pl.BlockSpec((_R, _DHF), lambda i: (i, 0)),
    ],
    out_shape=(
        jax.ShapeDtypeStruct((_N, _DH), jnp.float32),
        jax.ShapeDtypeStruct((_N, _DHF), jnp.float32),
        jax.ShapeDtypeStruct((_N, _DHF), jnp.float32),
    ),
)


def _final_body(o_ref, psum_ref, psq_ref, g_ref, be_ref, batch_ref,
                fcw_ref, fcb_ref, out_ref, xmax_s, ssum_s, cnt_s):
    i = pl.program_id(0)
    h = _bn_relu(o_ref[...], psum_ref[...], psq_ref[...], g_ref[...], be_ref[...])
    bm = batch_ref[...]
    seg = lax.broadcasted_iota(jnp.int32, (_R, _NB), 1)
    mf = (bm == seg).astype(jnp.float32)
    ones = jnp.ones((_R, 1), jnp.float32)
    ss = lax.dot_general(mf, h, (((0,), (0,)), ((), ())),
                         preferred_element_type=jnp.float32)
    cn = lax.dot_general(mf, ones, (((0,), (0,)), ((), ())),
                         preferred_element_type=jnp.float32)

    @pl.when(i == 0)
    def _():
        xmax_s[...] = jnp.zeros_like(xmax_s)
        ssum_s[...] = jnp.zeros_like(ssum_s)
        cnt_s[...] = jnp.zeros_like(cnt_s)

    ssum_s[...] += ss
    cnt_s[...] += cn

    lo = jnp.min(bm)
    hi = jnp.max(bm)
    segid = lax.broadcasted_iota(jnp.int32, (_NB, 1), 0)

    def seg_step(sgi, carry):
        col = (bm == sgi).astype(jnp.float32)
        v = jnp.max(h * col, axis=0, keepdims=True)
        oh = (segid == sgi).astype(jnp.float32)
        xmax_s[...] = jnp.maximum(xmax_s[...], oh * v)
        return carry

    lax.fori_loop(lo, hi + 1, seg_step, 0)

    @pl.when(i == _G - 1)
    def _():
        smean = ssum_s[...] / jnp.maximum(cnt_s[...], 1.0)
        gemb = jnp.concatenate([xmax_s[...], smean], axis=1)
        out_ref[...] = (jnp.dot(gemb, fcw_ref[...],
                                preferred_element_type=jnp.float32)
                        + fcb_ref[...])


_final_call = pl.pallas_call(
    _final_body,
    interpret=_IT,
    grid=(_G,),
    in_specs=[
        pl.BlockSpec((_R, _DH), lambda i: (i, 0)),
        pl.BlockSpec((1, _DH), lambda i: (0, 0)),
        pl.BlockSpec((1, _DH), lambda i: (0, 0)),
        pl.BlockSpec((1, _DH), lambda i: (0, 0)),
        pl.BlockSpec((1, _DH), lambda i: (0, 0)),
        pl.BlockSpec((_R, 1), lambda i: (i, 0)),
        pl.BlockSpec((2 * _DH, 2), lambda i: (0, 0)),
        pl.BlockSpec((1, 2), lambda i: (0, 0)),
    ],
    out_specs=pl.BlockSpec((_NB, 2), lambda i: (0, 0)),
    out_shape=jax.ShapeDtypeStruct((_NB, 2), jnp.float32),
    scratch_shapes=[
        pltpu.VMEM((_NB, _DH), jnp.float32),
        pltpu.VMEM((_NB, _DH), jnp.float32),
        pltpu.VMEM((_NB, 1), jnp.float32),
    ],
)


def _fmid_body(aggA_ref, aggB_ref, h1_ref, disA_ref, disB_ref, b_ref,
               g_ref, be_ref, w2_ref, h2_ref, hpA_ref, hpB_ref,
               o_scr, ps_s, pq_s):
    i = pl.program_id(0)

    @pl.when(i < _G)
    def _():
        d = disA_ref[...]
        o = (d * jnp.concatenate([aggA_ref[...], aggB_ref[...]], axis=1)
             + (d * d) * h1_ref[...] + b_ref[...])
        o_scr[pl.ds(i * _R, _R), :] = o

        @pl.when(i == 0)
        def _():
            ps_s[...] = jnp.zeros_like(ps_s)
            pq_s[...] = jnp.zeros_like(pq_s)

        ps_s[...] += jnp.sum(o, axis=0, keepdims=True)
        pq_s[...] += jnp.sum(o * o, axis=0, keepdims=True)

    @pl.when(i >= _G)
    def _():
        o = o_scr[pl.ds((i - _G) * _R, _R), :]
        h = _bn_relu(o, ps_s[...], pq_s[...], g_ref[...], be_ref[...])
        h2 = jnp.dot(h, w2_ref[...], preferred_element_type=jnp.float32)
        h2_ref[...] = h2
        hp = h2 * disB_ref[...]
        hpA_ref[...] = hp[:, :_DHF]
        hpB_ref[...] = hp[:, _DHF:]


_fmid_call = pl.pallas_call(
    _fmid_body,
    interpret=_IT,
    grid=(2 * _G,),
    in_specs=[
        pl.BlockSpec((_R, _DHF), lambda i: (jnp.minimum(i, _G - 1), 0)),
        pl.BlockSpec((_R, _DHF), lambda i: (jnp.minimum(i, _G - 1), 0)),
        pl.BlockSpec((_R, _DH), lambda i: (jnp.minimum(i, _G - 1), 0)),
        pl.BlockSpec((_R, 1), lambda i: (jnp.minimum(i, _G - 1), 0)),
        pl.BlockSpec((_R, 1), lambda i: (jnp.maximum(i - _G, 0), 0)),
        pl.BlockSpec((1, _DH), lambda i: (0, 0)),
        pl.BlockSpec((1, _DH), lambda i: (0, 0)),
        pl.BlockSpec((1, _DH), lambda i: (0, 0)),
        pl.BlockSpec((_DH, _DH), lambda i: (0, 0)),
    ],
    out_specs=[
        pl.BlockSpec((_R, _DH), lambda i: (jnp.maximum(i - _G, 0), 0)),
        pl.BlockSpec((_R, _DHF), lambda i: (jnp.maximum(i - _G, 0), 0)),
        pl.BlockSpec((_R, _DHF), lambda i: (jnp.maximum(i - _G, 0), 0)),
    ],
    out_shape=(
        jax.ShapeDtypeStruct((_N, _DH), jnp.float32),
        jax.ShapeDtypeStruct((_N, _DHF), jnp.float32),
        jax.ShapeDtypeStruct((_N, _DHF), jnp.float32),
    ),
    scratch_shapes=[
        pltpu.VMEM((_N, _DH), jnp.float32),
        pltpu.VMEM((1, _DH), jnp.float32),
        pltpu.VMEM((1, _DH), jnp.float32),
    ],
)


def _ffinal_body(aggA_ref, aggB_ref, h2_ref, disA_ref, b_ref, g_ref, be_ref,
                 batch_ref, fcw_ref, fcb_ref, out_ref,
                 o_scr, ps_s, pq_s, xmax_s, ssum_s, cnt_s):
    i = pl.program_id(0)

    @pl.when(i < _G)
    def _():
        d = disA_ref[...]
        o = (d * jnp.concatenate([aggA_ref[...], aggB_ref[...]], axis=1)
             + (d * d) * h2_ref[...] + b_ref[...])
        o_scr[pl.ds(i * _R, _R), :] = o

        @pl.when(i == 0)
        def _():
            ps_s[...] = jnp.zeros_like(ps_s)
            pq_s[...] = jnp.zeros_like(pq_s)

        ps_s[...] += jnp.sum(o, axis=0, keepdims=True)
        pq_s[...] += jnp.sum(o * o, axis=0, keepdims=True)

    @pl.when(i >= _G)
    def _():
        o = o_scr[pl.ds((i - _G) * _R, _R), :]
        h = _bn_relu(o, ps_s[...], pq_s[...], g_ref[...], be_ref[...])
        bm = batch_ref[...]
        seg = lax.broadcasted_iota(jnp.int32, (_R, _NB), 1)
        mf = (bm == seg).astype(jnp.float32)
        ones = jnp.ones((_R, 1), jnp.float32)
        ss = lax.dot_general(mf, h, (((0,), (0,)), ((), ())),
                             preferred_element_type=jnp.float32)
        cn = lax.dot_general(mf, ones, (((0,), (0,)), ((), ())),
                             preferred_element_type=jnp.float32)

        @pl.when(i == _G)
        def _():
            xmax_s[...] = jnp.zeros_like(xmax_s)
            ssum_s[...] = jnp.zeros_like(ssum_s)
            cnt_s[...] = jnp.zeros_like(cnt_s)

        ssum_s[...] += ss
        cnt_s[...] += cn

        lo = jnp.min(bm)
        hi = jnp.max(bm)
        segid = lax.broadcasted_iota(jnp.int32, (_NB, 1), 0)

        def seg_step(sgi, carry):
            col = (bm == sgi).astype(jnp.float32)
            v = jnp.max(h * col, axis=0, keepdims=True)
            oh = (segid == sgi).astype(jnp.float32)
            xmax_s[...] = jnp.maximum(xmax_s[...], oh * v)
            return carry

        lax.fori_loop(lo, hi + 1, seg_step, 0)

        @pl.when(i == 2 * _G - 1)
        def _():
            smean = ssum_s[...] / jnp.maximum(cnt_s[...], 1.0)
            gemb = jnp.concatenate([xmax_s[...], smean], axis=1)
            out_ref[...] = (jnp.dot(gemb, fcw_ref[...],
                                    preferred_element_type=jnp.float32)
                            + fcb_ref[...])


_ffinal_call = pl.pallas_call(
    _ffinal_body,
    interpret=_IT,
    grid=(2 * _G,),
    in_specs=[
        pl.BlockSpec((_R, _DHF), lambda i: (jnp.minimum(i, _G - 1), 0)),
        pl.BlockSpec((_R, _DHF), lambda i: (jnp.minimum(i, _G - 1), 0)),
        pl.BlockSpec((_R, _DH), lambda i: (jnp.minimum(i, _G - 1), 0)),
        pl.BlockSpec((_R, 1), lambda i: (jnp.minimum(i, _G - 1), 0)),
        pl.BlockSpec((1, _DH), lambda i: (0, 0)),
        pl.BlockSpec((1, _DH), lambda i: (0, 0)),
        pl.BlockSpec((1, _DH), lambda i: (0, 0)),
        pl.BlockSpec((_R, 1), lambda i: (jnp.maximum(i - _G, 0), 0)),
        pl.BlockSpec((2 * _DH, 2), lambda i: (0, 0)),
        pl.BlockSpec((1, 2), lambda i: (0, 0)),
    ],
    out_specs=pl.BlockSpec((_NB, 2), lambda i: (0, 0)),
    out_shape=jax.ShapeDtypeStruct((_NB, 2), jnp.float32),
    scratch_shapes=[
        pltpu.VMEM((_N, _DH), jnp.float32),
        pltpu.VMEM((1, _DH), jnp.float32),
        pltpu.VMEM((1, _DH), jnp.float32),
        pltpu.VMEM((_NB, _DH), jnp.float32),
        pltpu.VMEM((_NB, _DH), jnp.float32),
        pltpu.VMEM((_NB, 1), jnp.float32),
    ],
)


def kernel(x, edge_index, batch, W1, b1, g1, be1, W2, b2, g2, be2, fcW, fcb):
    es = edge_index[0].reshape(_NS, _AGG_CH, _K)
    ed = edge_index[1].reshape(_NS, _AGG_CH, _K)
    edd = edge_index[1].reshape(_NC * _NS, _DEG_CH, _K)
    zrows = jnp.zeros((_N, _DHF), jnp.float32)
    zvec = jnp.zeros((_N,), jnp.float32)

    degP = _deg_call(edd, zvec).reshape(_NC, _N)
    degT = degP.T

    h1, hpA, hpB, dis = _lin1_call(x, W1, degT)
    aggA, aggB = _agg_call(hpA, hpB, es, ed, zrows)
    h2, h2pA, h2pB = _fmid_call(aggA, aggB, h1, dis, dis, b1.reshape(1, -1),
                                g1.reshape(1, -1), be1.reshape(1, -1), W2)
    agg2A, agg2B = _agg_call(h2pA, h2pB, es, ed, zrows)
    out = _ffinal_call(agg2A, agg2B, h2, dis, b2.reshape(1, -1),
                       g2.reshape(1, -1), be2.reshape(1, -1),
                       batch.reshape(-1, 1), fcW, fcb.reshape(1, -1))
    return out


# speedup vs baseline: 24.0508x; 1.0028x over previous
import jax
import jax.numpy as jnp
from jax import lax
from jax.experimental import pallas as pl
from jax.experimental.pallas import tpu as pltpu
from jax.experimental.pallas import tpu_sc as plsc

_N = 10000
_E = 320000
_DIN = 128
_DH = 256
_DHF = 128
_NB = 64
_NC = 2
_NS = 16
_K = 100
_AGG_CH = _E // (_NS * _K)
_DEG_CH = _E // (_NC * _NS * _K)
_QCH = 40
_NPH = _AGG_CH // _QCH
_RPT = 624
_TAIL0 = _RPT * _NS
_TAILN = _N - _TAIL0

_R = 1000
_G = _N // _R

_mesh = plsc.VectorSubcoreMesh(core_axis_name="c", subcore_axis_name="s",
                               num_cores=_NC, num_subcores=_NS)


def _deg_body(e_dst, zvec, out, idx_v, ones_v, tmp_v, acc):
    c = lax.axis_index("c")
    s = lax.axis_index("s")
    wid = s * _NC + c
    pltpu.sync_copy(e_dst.at[wid], idx_v)
    for i in range(8):
        ones_v[pl.ds(i * 16, 16)] = jnp.ones((16,), jnp.float32)

    @pl.when(s == 0)
    def _():
        pltpu.sync_copy(zvec, tmp_v)
        pltpu.sync_copy(tmp_v, acc)

    plsc.subcore_barrier()

    def step(j, carry):
        pltpu.sync_copy(ones_v.at[pl.ds(0, _K)], acc.at[idx_v.at[j]], add=True)
        return carry

    lax.fori_loop(0, _DEG_CH, step, 0)
    plsc.subcore_barrier()

    @pl.when(s == 0)
    def _():
        pltpu.sync_copy(acc, tmp_v)
        pltpu.sync_copy(tmp_v, out.at[pl.ds(c * _N, _N)])


_deg_call = pl.kernel(
    _deg_body,
    out_type=jax.ShapeDtypeStruct((_NC * _N,), jnp.float32),
    mesh=_mesh,
    scratch_types=[
        pltpu.VMEM((_DEG_CH, _K), jnp.int32),
        pltpu.VMEM((128,), jnp.float32),
        pltpu.VMEM((_N,), jnp.float32),
        pltpu.VMEM_SHARED((_N,), jnp.float32),
    ],
)


def _agg_body(hpA, hpB, e_src, e_dst, zrows, outA, outB,
              sidx, didx, rbuf, acc, gsem, ssem):
    c = lax.axis_index("c")
    s = lax.axis_index("s")
    row0 = s * _RPT
    pltpu.sync_copy(zrows.at[pl.ds(row0, _RPT)], acc.at[pl.ds(row0, _RPT)])

    @pl.when(s == _NS - 1)
    def _():
        pltpu.sync_copy(zrows.at[pl.ds(_TAIL0, _TAILN)],
                        acc.at[pl.ds(_TAIL0, _TAILN)])

    plsc.subcore_barrier()

    def run(table):
        def gather(j):
            pltpu.async_copy(table.at[sidx.at[j]], rbuf.at[j % 3],
                             gsem.at[j % 3])

        def step(j, carry):
            pltpu.make_async_copy(table.at[sidx.at[j]], rbuf.at[j % 3],
                                  gsem.at[j % 3]).wait()
            pltpu.async_copy(rbuf.at[j % 3], acc.at[didx.at[j]],
                             ssem.at[j % 3], add=True)

            @pl.when(j >= 1)
            def _():
                pltpu.make_async_copy(rbuf.at[(j - 1) % 3],
                                      acc.at[didx.at[j - 1]],
                                      ssem.at[(j - 1) % 3]).wait()

            @pl.when(j + 2 < _QCH)
            def _():
                gather(j + 2)

            return carry

        for p in range(_NPH):
            pltpu.sync_copy(e_src.at[s, pl.ds(p * _QCH, _QCH)], sidx)
            pltpu.sync_copy(e_dst.at[s, pl.ds(p * _QCH, _QCH)], didx)
            gather(0)
            gather(1)
            lax.fori_loop(0, _QCH, step, 0)
            pltpu.make_async_copy(rbuf.at[(_QCH - 1) % 3],
                                  acc.at[didx.at[_QCH - 1]],
                                  ssem.at[(_QCH - 1) % 3]).wait()

    @pl.when(c == 0)
    def _():
        run(hpA)

    @pl.when(c == 1)
    def _():
        run(hpB)

    plsc.subcore_barrier()

    def copy_out(outref):
        pltpu.sync_copy(acc.at[pl.ds(row0, _RPT)], outref.at[pl.ds(row0, _RPT)])

        @pl.when(s == _NS - 1)
        def _():
            pltpu.sync_copy(acc.at[pl.ds(_TAIL0, _TAILN)],
                            outref.at[pl.ds(_TAIL0, _TAILN)])

    @pl.when(c == 0)
    def _():
        copy_out(outA)

    @pl.when(c == 1)
    def _():
        copy_out(outB)


_agg_call = pl.kernel(
    _agg_body,
    out_type=(jax.ShapeDtypeStruct((_N, _DHF), jnp.float32),
              jax.ShapeDtypeStruct((_N, _DHF), jnp.float32)),
    mesh=_mesh,
    scratch_types=[
        pltpu.VMEM((_QCH, _K), jnp.int32),
        pltpu.VMEM((_QCH, _K), jnp.int32),
        pltpu.VMEM((3, _K, _DHF), jnp.float32),
        pltpu.VMEM_SHARED((_N, _DHF), jnp.float32),
        pltpu.SemaphoreType.DMA((3,)),
        pltpu.SemaphoreType.DMA((3,)),
    ],
)


def _lin1_body(x_ref, w1_ref, degT_ref, h1_ref, hpA_ref, hpB_ref, dis_ref):
    deg = degT_ref[:, 0:1] + degT_ref[:, 1:2] + 1.0
    dis = lax.rsqrt(deg)
    h1 = jnp.dot(x_ref[...], w1_ref[...], preferred_element_type=jnp.float32)
    hp = h1 * dis
    h1_ref[...] = h1
    hpA_ref[...] = hp[:, :_DHF]
    hpB_ref[...] = hp[:, _DHF:]
    dis_ref[...] = dis


_lin1_call = pl.pallas_call(
    _lin1_body,
    grid=(_G,),
    in_specs=[
        pl.BlockSpec((_R, _DIN), lambda i: (i, 0)),
        pl.BlockSpec((_DIN, _DH), lambda i: (0, 0)),
        pl.BlockSpec((_R, 2), lambda i: (i, 0)),
    ],
    out_specs=[
        pl.BlockSpec((_R, _DH), lambda i: (i, 0)),
        pl.BlockSpec((_R, _DHF), lambda i: (i, 0)),
        pl.BlockSpec((_R, _DHF), lambda i: (i, 0)),
        pl.BlockSpec((_R, 1), lambda i: (i, 0)),
    ],
    out_shape=(
        jax.ShapeDtypeStruct((_N, _DH), jnp.float32),
        jax.ShapeDtypeStruct((_N, _DHF), jnp.float32),
        jax.ShapeDtypeStruct((_N, _DHF), jnp.float32),
        jax.ShapeDtypeStruct((_N, 1), jnp.float32),
    ),
)


def _bn_relu(o, psum, psq, g, be):
    mu = psum * (1.0 / _N)
    var = psq * (1.0 / _N) - mu * mu
    return jnp.maximum(g * (o - mu) * lax.rsqrt(var + 1e-5) + be, 0.0)


def _fmid_body(aggA_ref, aggB_ref, h1_ref, disA_ref, disB_ref, b_ref,
               g_ref, be_ref, w2_ref, h2_ref, hpA_ref, hpB_ref,
               o_scr, ps_s, pq_s):
    i = pl.program_id(0)

    @pl.when(i < _G)
    def _():
        d = disA_ref[...]
        o = (d * jnp.concatenate([aggA_ref[...], aggB_ref[...]], axis=1)
             + (d * d) * h1_ref[...] + b_ref[...])
        o_scr[pl.ds(i * _R, _R), :] = o

        @pl.when(i == 0)
        def _():
            ps_s[...] = jnp.zeros_like(ps_s)
            pq_s[...] = jnp.zeros_like(pq_s)

        ps_s[...] += jnp.sum(o, axis=0, keepdims=True)
        pq_s[...] += jnp.sum(o * o, axis=0, keepdims=True)

    @pl.when(i >= _G)
    def _():
        o = o_scr[pl.ds((i - _G) * _R, _R), :]
        h = _bn_relu(o, ps_s[...], pq_s[...], g_ref[...], be_ref[...])
        h2 = jnp.dot(h, w2_ref[...], preferred_element_type=jnp.float32)
        h2_ref[...] = h2
        hp = h2 * disB_ref[...]
        hpA_ref[...] = hp[:, :_DHF]
        hpB_ref[...] = hp[:, _DHF:]


_fmid_call = pl.pallas_call(
    _fmid_body,
    grid=(2 * _G,),
    in_specs=[
        pl.BlockSpec((_R, _DHF), lambda i: (jnp.minimum(i, _G - 1), 0)),
        pl.BlockSpec((_R, _DHF), lambda i: (jnp.minimum(i, _G - 1), 0)),
        pl.BlockSpec((_R, _DH), lambda i: (jnp.minimum(i, _G - 1), 0)),
        pl.BlockSpec((_R, 1), lambda i: (jnp.minimum(i, _G - 1), 0)),
        pl.BlockSpec((_R, 1), lambda i: (jnp.maximum(i - _G, 0), 0)),
        pl.BlockSpec((1, _DH), lambda i: (0, 0)),
        pl.BlockSpec((1, _DH), lambda i: (0, 0)),
        pl.BlockSpec((1, _DH), lambda i: (0, 0)),
        pl.BlockSpec((_DH, _DH), lambda i: (0, 0)),
    ],
    out_specs=[
        pl.BlockSpec((_R, _DH), lambda i: (jnp.maximum(i - _G, 0), 0)),
        pl.BlockSpec((_R, _DHF), lambda i: (jnp.maximum(i - _G, 0), 0)),
        pl.BlockSpec((_R, _DHF), lambda i: (jnp.maximum(i - _G, 0), 0)),
    ],
    out_shape=(
        jax.ShapeDtypeStruct((_N, _DH), jnp.float32),
        jax.ShapeDtypeStruct((_N, _DHF), jnp.float32),
        jax.ShapeDtypeStruct((_N, _DHF), jnp.float32),
    ),
    scratch_shapes=[
        pltpu.VMEM((_N, _DH), jnp.float32),
        pltpu.VMEM((1, _DH), jnp.float32),
        pltpu.VMEM((1, _DH), jnp.float32),
    ],
)


def _ffinal_body(aggA_ref, aggB_ref, h2_ref, disA_ref, b_ref, g_ref, be_ref,
                 batch_ref, fcw_ref, fcb_ref, out_ref,
                 o_scr, ps_s, pq_s, xmax_s, ssum_s, cnt_s):
    i = pl.program_id(0)

    @pl.when(i < _G)
    def _():
        d = disA_ref[...]
        o = (d * jnp.concatenate([aggA_ref[...], aggB_ref[...]], axis=1)
             + (d * d) * h2_ref[...] + b_ref[...])
        o_scr[pl.ds(i * _R, _R), :] = o

        @pl.when(i == 0)
        def _():
            ps_s[...] = jnp.zeros_like(ps_s)
            pq_s[...] = jnp.zeros_like(pq_s)

        ps_s[...] += jnp.sum(o, axis=0, keepdims=True)
        pq_s[...] += jnp.sum(o * o, axis=0, keepdims=True)

    @pl.when(i >= _G)
    def _():
        o = o_scr[pl.ds((i - _G) * _R, _R), :]
        h = _bn_relu(o, ps_s[...], pq_s[...], g_ref[...], be_ref[...])
        bm = batch_ref[...]
        seg = lax.broadcasted_iota(jnp.int32, (_R, _NB), 1)
        mf = (bm == seg).astype(jnp.float32)
        ones = jnp.ones((_R, 1), jnp.float32)
        ss = lax.dot_general(mf, h, (((0,), (0,)), ((), ())),
                             preferred_element_type=jnp.float32)
        cn = lax.dot_general(mf, ones, (((0,), (0,)), ((), ())),
                             preferred_element_type=jnp.float32)

        @pl.when(i == _G)
        def _():
            xmax_s[...] = jnp.zeros_like(xmax_s)
            ssum_s[...] = jnp.zeros_like(ssum_s)
            cnt_s[...] = jnp.zeros_like(cnt_s)

        ssum_s[...] += ss
        cnt_s[...] += cn

        lo = jnp.min(bm)
        hi = jnp.max(bm)
        segid = lax.broadcasted_iota(jnp.int32, (_NB, 1), 0)

        def seg_step(sgi, carry):
            col = (bm == sgi).astype(jnp.float32)
            v = jnp.max(h * col, axis=0, keepdims=True)
            oh = (segid == sgi).astype(jnp.float32)
            xmax_s[...] = jnp.maximum(xmax_s[...], oh * v)
            return carry

        lax.fori_loop(lo, hi + 1, seg_step, 0)

        @pl.when(i == 2 * _G - 1)
        def _():
            smean = ssum_s[...] / jnp.maximum(cnt_s[...], 1.0)
            gemb = jnp.concatenate([xmax_s[...], smean], axis=1)
            out_ref[...] = (jnp.dot(gemb, fcw_ref[...],
                                    preferred_element_type=jnp.float32)
                            + fcb_ref[...])


_ffinal_call = pl.pallas_call(
    _ffinal_body,
    grid=(2 * _G,),
    in_specs=[
        pl.BlockSpec((_R, _DHF), lambda i: (jnp.minimum(i, _G - 1), 0)),
        pl.BlockSpec((_R, _DHF), lambda i: (jnp.minimum(i, _G - 1), 0)),
        pl.BlockSpec((_R, _DH), lambda i: (jnp.minimum(i, _G - 1), 0)),
        pl.BlockSpec((_R, 1), lambda i: (jnp.minimum(i, _G - 1), 0)),
        pl.BlockSpec((1, _DH), lambda i: (0, 0)),
        pl.BlockSpec((1, _DH), lambda i: (0, 0)),
        pl.BlockSpec((1, _DH), lambda i: (0, 0)),
        pl.BlockSpec((_R, 1), lambda i: (jnp.maximum(i - _G, 0), 0)),
        pl.BlockSpec((2 * _DH, 2), lambda i: (0, 0)),
        pl.BlockSpec((1, 2), lambda i: (0, 0)),
    ],
    out_specs=pl.BlockSpec((_NB, 2), lambda i: (0, 0)),
    out_shape=jax.ShapeDtypeStruct((_NB, 2), jnp.float32),
    scratch_shapes=[
        pltpu.VMEM((_N, _DH), jnp.float32),
        pltpu.VMEM((1, _DH), jnp.float32),
        pltpu.VMEM((1, _DH), jnp.float32),
        pltpu.VMEM((_NB, _DH), jnp.float32),
        pltpu.VMEM((_NB, _DH), jnp.float32),
        pltpu.VMEM((_NB, 1), jnp.float32),
    ],
)


def kernel(x, edge_index, batch, W1, b1, g1, be1, W2, b2, g2, be2, fcW, fcb):
    es = edge_index[0].reshape(_NS, _AGG_CH, _K)
    ed = edge_index[1].reshape(_NS, _AGG_CH, _K)
    edd = edge_index[1].reshape(_NC * _NS, _DEG_CH, _K)
    zrows = jnp.zeros((_N, _DHF), jnp.float32)
    zvec = jnp.zeros((_N,), jnp.float32)

    degP = _deg_call(edd, zvec).reshape(_NC, _N)
    degT = degP.T

    h1, hpA, hpB, dis = _lin1_call(x, W1, degT)
    aggA, aggB = _agg_call(hpA, hpB, es, ed, zrows)
    h2, h2pA, h2pB = _fmid_call(aggA, aggB, h1, dis, dis, b1.reshape(1, -1),
                                g1.reshape(1, -1), be1.reshape(1, -1), W2)
    agg2A, agg2B = _agg_call(h2pA, h2pB, es, ed, zrows)
    out = _ffinal_call(agg2A, agg2B, h2, dis, b2.reshape(1, -1),
                       g2.reshape(1, -1), be2.reshape(1, -1),
                       batch.reshape(-1, 1), fcW, fcb.reshape(1, -1))
    return out
